# baseline (speedup 1.0000x reference)
import functools

import jax
import jax.numpy as jnp
from jax import lax
from jax.experimental import pallas as pl
from jax.experimental.pallas import tpu as pltpu
from jax.experimental.pallas import tpu_sc as plsc

_VOCAB = 100000
_CTX = 2048
_DIM = 1024
_BATCH = 4

_NC = 2
_NS = 16
_NW = _NC * _NS
_PW = _CTX // _NW
_SUB = 8
_NSTEP = _PW // _SUB
_STEPS = _BATCH * _NSTEP
_NBUF = 6
_GLEAD = 4
_LANES = 16
_CGRP = 32


def _body(x_hbm, tok_hbm, pos_hbm, out_hbm, idx_v, pos_v, *ring):
    rows = ring[:_NBUF]
    gsem = ring[_NBUF:2 * _NBUF]
    osem = ring[2 * _NBUF:]

    wid = lax.axis_index("s") * _NC + lax.axis_index("c")
    p0 = wid * _PW

    for b in range(_BATCH):
        pltpu.sync_copy(x_hbm.at[b, pl.ds(p0, _PW)], idx_v.at[b])
    pltpu.sync_copy(pos_hbm.at[pl.ds(p0, _PW)], pos_v)

    gd = {}
    od = {}

    def gather(s):
        b, c = divmod(s, _NSTEP)
        gd[s] = pltpu.async_copy(
            tok_hbm.at[idx_v.at[b, pl.ds(c * _SUB, _SUB)]],
            rows[s % _NBUF], gsem[s % _NBUF])

    def outcopy(s):
        b, c = divmod(s, _NSTEP)
        od[s] = pltpu.async_copy(
            rows[s % _NBUF],
            out_hbm.at[b, pl.ds(p0 + c * _SUB, _SUB)], osem[s % _NBUF])

    def add_pos(s):
        c = s % _NSTEP
        buf = rows[s % _NBUF]
        ngrp = _DIM // _LANES // _CGRP

        def add_row(r, _):
            def add_grp(g, _):
                for d in range(_CGRP):
                    sl = pl.ds(g * (_CGRP * _LANES) + d * _LANES, _LANES)
                    plsc.addupdate(buf.at[r, sl], pos_v[c * _SUB + r, sl])
                return 0

            lax.fori_loop(0, ngrp, add_grp, 0)
            return 0

        lax.fori_loop(0, _SUB, add_row, 0)

    for s in range(_GLEAD):
        gather(s)
    for s in range(_STEPS):
        kg = s + _GLEAD
        if kg < _STEPS:
            if kg >= _NBUF:
                od[kg - _NBUF].wait()
            gather(kg)
        gd[s].wait()
        add_pos(s)
        outcopy(s)
    for s in range(_STEPS - _NBUF, _STEPS):
        od[s].wait()


def kernel(x, token_table, pos_table):
    mesh = plsc.VectorSubcoreMesh(core_axis_name="c", subcore_axis_name="s")
    run = functools.partial(
        pl.kernel,
        mesh=mesh,
        out_type=jax.ShapeDtypeStruct((_BATCH, _CTX, _DIM), jnp.float32),
        scratch_types=(
            [pltpu.VMEM((_BATCH, _PW), jnp.int32),
             pltpu.VMEM((_PW, _DIM), jnp.float32)]
            + [pltpu.VMEM((_SUB, _DIM), jnp.float32)] * _NBUF
            + [pltpu.SemaphoreType.DMA] * (2 * _NBUF)
        ),
    )(_body)
    return run(x, token_table, pos_table)


# speedup vs baseline: 1.3911x; 1.3911x over previous
import functools

import jax
import jax.numpy as jnp
from jax import lax
from jax.experimental import pallas as pl
from jax.experimental.pallas import tpu as pltpu
from jax.experimental.pallas import tpu_sc as plsc

_VOCAB = 100000
_CTX = 2048
_DIM = 1024
_BATCH = 4

_NC = 2
_NS = 16
_NW = _NC * _NS
_PW = _CTX // _NW
_SUB = 32
_STEPS = 2 * _BATCH
_NBUF = 2
_LANES = 16


def _body(x_hbm, tok_hbm, pos_hbm, out_hbm, idx_v, pos_v, rows0, rows1,
          gs0, gs1, os0, os1):
    rows = [rows0, rows1]
    gsem = [gs0, gs1]
    osem = [os0, os1]

    wid = lax.axis_index("s") * _NC + lax.axis_index("c")
    p0 = wid * _PW

    for b in range(_BATCH):
        pltpu.sync_copy(x_hbm.at[b, pl.ds(p0, _PW)], idx_v.at[b])

    gd = {}
    od = {}

    def gather(s):
        h, b = divmod(s, _BATCH)
        gd[s] = pltpu.async_copy(
            tok_hbm.at[idx_v.at[b, pl.ds(h * _SUB, _SUB)]],
            rows[s % _NBUF], gsem[s % _NBUF])

    def outcopy(s):
        h, b = divmod(s, _BATCH)
        od[s] = pltpu.async_copy(
            rows[s % _NBUF],
            out_hbm.at[b, pl.ds(p0 + h * _SUB, _SUB)], osem[s % _NBUF])

    def stage_pos(h):
        pltpu.sync_copy(pos_hbm.at[pl.ds(p0 + h * _SUB, _SUB)], pos_v)

    def add_pos(s):
        buf = rows[s % _NBUF]

        def add_row(r, _):
            for d in range(_DIM // _LANES):
                sl = pl.ds(d * _LANES, _LANES)
                plsc.addupdate(buf.at[r, sl], pos_v[r, sl])
            return 0

        lax.fori_loop(0, _SUB, add_row, 0)

    stage_pos(0)
    gather(0)
    gather(1)
    for s in range(_STEPS):
        if s >= 1 and s + 1 < _STEPS:
            od[s - 1].wait()
            gather(s + 1)
        gd[s].wait()
        add_pos(s)
        outcopy(s)
        if s == _BATCH - 1:
            stage_pos(1)
    od[_STEPS - 2].wait()
    od[_STEPS - 1].wait()


def kernel(x, token_table, pos_table):
    mesh = plsc.VectorSubcoreMesh(core_axis_name="c", subcore_axis_name="s")
    run = functools.partial(
        pl.kernel,
        mesh=mesh,
        out_type=jax.ShapeDtypeStruct((_BATCH, _CTX, _DIM), jnp.float32),
        scratch_types=[
            pltpu.VMEM((_BATCH, _PW), jnp.int32),
            pltpu.VMEM((_SUB, _DIM), jnp.float32),
            pltpu.VMEM((_SUB, _DIM), jnp.float32),
            pltpu.VMEM((_SUB, _DIM), jnp.float32),
            pltpu.SemaphoreType.DMA,
            pltpu.SemaphoreType.DMA,
            pltpu.SemaphoreType.DMA,
            pltpu.SemaphoreType.DMA,
        ],
    )(_body)
    return run(x, token_table, pos_table)


# speedup vs baseline: 1.4158x; 1.0177x over previous
import functools

import jax
import jax.numpy as jnp
from jax import lax
from jax.experimental import pallas as pl
from jax.experimental.pallas import tpu as pltpu
from jax.experimental.pallas import tpu_sc as plsc

_VOCAB = 100000
_CTX = 2048
_DIM = 1024
_BATCH = 4

_NC = 2
_NS = 16
_NW = _NC * _NS
_PW = _CTX // _NW
_SUB = 32
_STEPS = 2 * _BATCH
_NBUF = 2
_LANES = 16


def _body(x_hbm, tok_hbm, pos_hbm, out_hbm, idx_v, pos_v, rows0, rows1,
          gs0, gs1, os0, os1, psem):
    rows = [rows0, rows1]
    gsem = [gs0, gs1]
    osem = [os0, os1]

    wid = lax.axis_index("s") * _NC + lax.axis_index("c")
    p0 = wid * _PW

    gd = {}
    od = {}

    def gather(s):
        h, b = divmod(s, _BATCH)
        gd[s] = pltpu.async_copy(
            tok_hbm.at[idx_v.at[b, pl.ds(h * _SUB, _SUB)]],
            rows[s % _NBUF], gsem[s % _NBUF])

    def outcopy(s):
        h, b = divmod(s, _BATCH)
        od[s] = pltpu.async_copy(
            rows[s % _NBUF],
            out_hbm.at[b, pl.ds(p0 + h * _SUB, _SUB)], osem[s % _NBUF])

    def stage_pos(h):
        return pltpu.async_copy(
            pos_hbm.at[pl.ds(p0 + h * _SUB, _SUB)], pos_v, psem)

    def add_pos(s):
        buf = rows[s % _NBUF]

        def add_row(r, _):
            for d in range(_DIM // _LANES):
                sl = pl.ds(d * _LANES, _LANES)
                plsc.addupdate(buf.at[r, sl], pos_v[r, sl])
            return 0

        lax.fori_loop(0, _SUB, add_row, 0)

    pltpu.sync_copy(x_hbm.at[0, pl.ds(p0, _PW)], idx_v.at[0])
    gather(0)
    pltpu.sync_copy(x_hbm.at[1, pl.ds(p0, _PW)], idx_v.at[1])
    gather(1)
    pd = stage_pos(0)
    pltpu.sync_copy(x_hbm.at[2, pl.ds(p0, _PW)], idx_v.at[2])
    pltpu.sync_copy(x_hbm.at[3, pl.ds(p0, _PW)], idx_v.at[3])
    for s in range(_STEPS):
        if s >= 1 and s + 1 < _STEPS:
            od[s - 1].wait()
            gather(s + 1)
        gd[s].wait()
        if s == 0 or s == _BATCH:
            pd.wait()
        add_pos(s)
        outcopy(s)
        if s == _BATCH - 1:
            pd = stage_pos(1)
    od[_STEPS - 2].wait()
    od[_STEPS - 1].wait()


def kernel(x, token_table, pos_table):
    mesh = plsc.VectorSubcoreMesh(core_axis_name="c", subcore_axis_name="s")
    run = functools.partial(
        pl.kernel,
        mesh=mesh,
        out_type=jax.ShapeDtypeStruct((_BATCH, _CTX, _DIM), jnp.float32),
        scratch_types=[
            pltpu.VMEM((_BATCH, _PW), jnp.int32),
            pltpu.VMEM((_SUB, _DIM), jnp.float32),
            pltpu.VMEM((_SUB, _DIM), jnp.float32),
            pltpu.VMEM((_SUB, _DIM), jnp.float32),
            pltpu.SemaphoreType.DMA,
            pltpu.SemaphoreType.DMA,
            pltpu.SemaphoreType.DMA,
            pltpu.SemaphoreType.DMA,
            pltpu.SemaphoreType.DMA,
        ],
    )(_body)
    return run(x, token_table, pos_table)
